# baseline (speedup 1.0000x reference)
import functools

import jax
import jax.numpy as jnp
from jax import lax
from jax.experimental import pallas as pl
from jax.experimental.pallas import tpu as pltpu
from jax.experimental.pallas import tpu_sc as plsc

VOCAB = 100000
EMBED = 64
B = 1024
CTX = 10

NC = 2
NS = 16
NW = NC * NS
EX_PER_W = B // NW
IDX_PER_W = EX_PER_W * CTX

VB = 2048
NV = (VOCAB + VB - 1) // VB
NEG_BIG = -1e30


def _sc_body(idx_hbm, table_hbm, out_hbm, idx_v, rows_v, out_v, sem):
    wid = lax.axis_index("s") * NC + lax.axis_index("c")
    base = wid * IDX_PER_W
    pltpu.sync_copy(idx_hbm.at[pl.ds(base, IDX_PER_W)], idx_v)
    pltpu.async_copy(table_hbm.at[idx_v], rows_v, sem).wait()

    def pool_one(e, _):
        r0 = e * CTX
        for c in range(EMBED // 16):
            sl = pl.ds(c * 16, 16)
            acc = rows_v[r0, sl]
            for j in range(1, CTX):
                acc = acc + rows_v[r0 + j, sl]
            out_v[e, sl] = acc * (1.0 / CTX)
        return 0

    lax.fori_loop(0, EX_PER_W, pool_one, 0)
    pltpu.sync_copy(out_v, out_hbm.at[pl.ds(wid * EX_PER_W, EX_PER_W)])


@functools.cache
def _make_gather_mean():
    return pl.kernel(
        _sc_body,
        out_type=jax.ShapeDtypeStruct((B, EMBED), jnp.float32),
        mesh=plsc.VectorSubcoreMesh(core_axis_name="c", subcore_axis_name="s"),
        compiler_params=pltpu.CompilerParams(use_tc_tiling_on_sc=False),
        scratch_types=[
            pltpu.VMEM((IDX_PER_W,), jnp.int32),
            pltpu.VMEM((IDX_PER_W, EMBED), jnp.float32),
            pltpu.VMEM((EX_PER_W, EMBED), jnp.float32),
            pltpu.SemaphoreType.DMA,
        ],
    )


def _stats_body(avg_ref, w_ref, lse_ref, m_ref, s_ref):
    j = pl.program_id(0)

    @pl.when(j == 0)
    def _init():
        m_ref[...] = jnp.full((B, 1), NEG_BIG, jnp.float32)
        s_ref[...] = jnp.zeros((B, 1), jnp.float32)

    logits = jnp.dot(avg_ref[...], w_ref[...], preferred_element_type=jnp.float32)
    cols = j * VB + lax.broadcasted_iota(jnp.int32, (1, VB), 1)
    logits = jnp.where(cols < VOCAB, logits, NEG_BIG)

    m_old = m_ref[...]
    m_new = jnp.maximum(m_old, jnp.max(logits, axis=1, keepdims=True))
    s_ref[...] = (s_ref[...] * jnp.exp(m_old - m_new)
                  + jnp.sum(jnp.exp(logits - m_new), axis=1, keepdims=True))
    m_ref[...] = m_new

    @pl.when(j == NV - 1)
    def _fini():
        lse_ref[...] = m_ref[...] + jnp.log(s_ref[...])


def _stats(avg, w):
    return pl.pallas_call(
        _stats_body,
        grid=(NV,),
        in_specs=[
            pl.BlockSpec((B, EMBED), lambda j: (0, 0)),
            pl.BlockSpec((EMBED, VB), lambda j: (0, j)),
        ],
        out_specs=pl.BlockSpec((B, 1), lambda j: (0, 0)),
        out_shape=jax.ShapeDtypeStruct((B, 1), jnp.float32),
        scratch_shapes=[
            pltpu.VMEM((B, 1), jnp.float32),
            pltpu.VMEM((B, 1), jnp.float32),
        ],
    )(avg, w)


def _softmax_body(avg_ref, w_ref, b_ref, lse_ref, out_ref):
    logits = jnp.dot(avg_ref[...], w_ref[...], preferred_element_type=jnp.float32)
    out_ref[...] = jnp.exp(logits + b_ref[...] - lse_ref[...])


def _softmax(avg, w, b2d, lse):
    return pl.pallas_call(
        _softmax_body,
        grid=(NV,),
        in_specs=[
            pl.BlockSpec((B, EMBED), lambda j: (0, 0)),
            pl.BlockSpec((EMBED, VB), lambda j: (0, j)),
            pl.BlockSpec((1, VB), lambda j: (0, j)),
            pl.BlockSpec((B, 1), lambda j: (0, 0)),
        ],
        out_specs=pl.BlockSpec((B, VB), lambda j: (0, j)),
        out_shape=jax.ShapeDtypeStruct((B, VOCAB), jnp.float32),
    )(avg, w, b2d, lse)


def kernel(context, emb_table, W, b):
    idx = context.reshape(-1).astype(jnp.int32)
    avg = _make_gather_mean()(idx, emb_table)
    lse = _stats(avg, W)
    return _softmax(avg, W, b.reshape(1, VOCAB), lse)
